# baseline (speedup 1.0000x reference)
import functools

import jax
import jax.numpy as jnp
from jax import lax
from jax.experimental import pallas as pl
from jax.experimental.pallas import tpu as pltpu
from jax.experimental.pallas import tpu_sc as plsc

D = 128
N_WORKERS = 32
CHUNK = 200
NBUF = 4
LA = 2


def _emb_kernel(n_total):
    per_w = n_total // N_WORKERS
    n_chunks = per_w // CHUNK
    mesh = plsc.VectorSubcoreMesh(core_axis_name="c", subcore_axis_name="s")

    @functools.partial(
        pl.kernel,
        mesh=mesh,
        out_type=jax.ShapeDtypeStruct((n_total, D), jnp.float32),
        scratch_types=[
            pltpu.VMEM((CHUNK,), jnp.int32),
            pltpu.VMEM((CHUNK,), jnp.int32),
            pltpu.VMEM((CHUNK,), jnp.int32),
            pltpu.VMEM((CHUNK,), jnp.int32),
            pltpu.VMEM((NBUF, CHUNK, D), jnp.float32),
            pltpu.SemaphoreType.DMA,
            pltpu.SemaphoreType.DMA,
            pltpu.SemaphoreType.DMA,
            pltpu.SemaphoreType.DMA,
            pltpu.SemaphoreType.DMA,
            pltpu.SemaphoreType.DMA,
            pltpu.SemaphoreType.DMA,
            pltpu.SemaphoreType.DMA,
        ],
    )
    def k(idx_hbm, tbl_hbm, out_hbm,
          i0, i1, i2, i3, rows_v,
          g0, g1, g2, g3, s0, s1, s2, s3):
        idxb = (i0, i1, i2, i3)
        gsem = (g0, g1, g2, g3)
        ssem = (s0, s1, s2, s3)
        wid = lax.axis_index("s") * 2 + lax.axis_index("c")
        base = wid * per_w

        def start_gather(c, b):
            pltpu.sync_copy(idx_hbm.at[pl.ds(base + c * CHUNK, CHUNK)], idxb[b])
            pltpu.async_copy(tbl_hbm.at[idxb[b]], rows_v.at[b], gsem[b])

        for c in range(LA):
            start_gather(c, c % NBUF)

        def body(g, carry):
            for b0 in range(NBUF):
                c = g * NBUF + b0
                b = b0
                pltpu.make_async_copy(
                    tbl_hbm.at[idxb[b]], rows_v.at[b], gsem[b]
                ).wait()
                out_slc = out_hbm.at[pl.ds(base + c * CHUNK, CHUNK)]
                pltpu.async_copy(rows_v.at[b], out_slc, ssem[b])

                nb = (b0 + LA) % NBUF

                @pl.when(c + LA < n_chunks)
                def _():
                    pc = c + LA - NBUF
                    @pl.when(pc >= 0)
                    def _():
                        prev = out_hbm.at[pl.ds(base + pc * CHUNK, CHUNK)]
                        pltpu.make_async_copy(
                            rows_v.at[nb], prev, ssem[nb]
                        ).wait()
                    start_gather(c + LA, nb)

            return carry

        lax.fori_loop(0, n_chunks // NBUF, body, 0)

        for c in range(n_chunks - NBUF, n_chunks):
            b = c % NBUF
            out_slc = out_hbm.at[pl.ds(base + c * CHUNK, CHUNK)]
            pltpu.make_async_copy(rows_v.at[b], out_slc, ssem[b]).wait()

    return k


def kernel(x, weight):
    b, s = x.shape
    n_total = b * s
    idx = x.reshape(n_total).astype(jnp.int32)
    out = _emb_kernel(n_total)(idx, weight)
    return out.reshape(b, s, weight.shape[1])
